# baseline (speedup 1.0000x reference)
import jax
import jax.numpy as jnp
import numpy as np
from jax.experimental import pallas as pl
from jax.experimental.pallas import tpu as pltpu

B, P, K_NN = 256, 128, 16
NUM_NODE_TYPE, HID, OUT, N_CLASSES = 34, 70, 70, 5
KERNEL, DIM, N_LAYERS = 3, 2, 4
N = B * P
JG = 4
NG = B // JG
PG = JG * P
F32 = jnp.float32
BF16 = jnp.bfloat16
BIG = np.float32(3.0e38)
INV_K = np.float32(1.0 / K_NN)
UG = 2


def _knn_one_t(ptst):
    xr = ptst[0:1, :]
    yr = ptst[1:2, :]
    xrow = jnp.broadcast_to(xr, (P, P))
    yrow = jnp.broadcast_to(yr, (P, P))
    xcol = jnp.transpose(xrow)
    ycol = jnp.transpose(yrow)
    dx = xcol - xrow
    dy = ycol - yrow
    score = dx * dx + dy * dy
    row = jax.lax.broadcasted_iota(jnp.int32, (P, P), 0)

    def body(_, carry):
        score, acc = carry
        m = jnp.min(score, axis=0, keepdims=True)
        cidx = jnp.where(score == m, row, np.int32(2 ** 30))
        sel = jnp.min(cidx, axis=0, keepdims=True)
        pick = row == sel
        acc = acc + pick.astype(F32)
        score = jnp.where(pick, BIG, score)
        return score, acc

    _, acc = jax.lax.fori_loop(0, K_NN, body, (score, jnp.zeros_like(score)))
    return acc


def _layer_w(wp, bp, mu, iv):
    ones = (jax.lax.broadcasted_iota(jnp.int32, (1, DIM), 0) * 0 + 1).astype(F32)
    c = np.float32(1.0) / jnp.sqrt(ones * np.float32(K_NN + 1.0))
    ps = jnp.dot(c, wp, preferred_element_type=F32)
    pp = jnp.tanh(ps + bp)
    d = pp - mu
    gk = jnp.sum((np.float32(-0.5) * (d * d)) * (iv * iv),
                 axis=1, keepdims=True)
    return jnp.exp(gk)


def _dot_t(lhs, rhs):
    return jax.lax.dot_general(lhs, rhs, (((0,), (0,)), ((), ())),
                               preferred_element_type=F32)


def _agg_group_t(a_scr, g, h4, fcw_ref, w, layer):
    hkw = None
    for k in range(KERNEL):
        fck = fcw_ref[layer * KERNEL + k]
        hkt4 = _dot_t(fck, h4)
        t = hkt4 * w[k:k + 1, 0:1]
        hkw = t if hkw is None else hkw + t
    hi4 = hkw.astype(BF16)
    mid4 = (hkw - hi4.astype(F32)).astype(BF16)
    yjs = []
    for u in range(JG):
        sl = slice(u * P, (u + 1) * P)
        himid = jnp.concatenate([hi4[:, sl], mid4[:, sl]], axis=1)
        at2 = a_scr[g * JG + u]
        aggt = jnp.dot(himid, at2, preferred_element_type=F32)
        yjs.append(aggt * INV_K)
    return jnp.concatenate(yjs, axis=1)


def _monet_krn(ptst_ref, featt_ref, wemb_ref, bembt_ref, wp_ref, bp_ref,
               mu_ref, is_ref, fcw_ref, gamt_ref, bett_ref, hg_ref,
               a_scr, h_scr, y_scr):
    def knn_body(g, _):
        for u in range(JG):
            i = g * JG + u
            at = _knn_one_t(ptst_ref[i]).astype(BF16)
            a_scr[i, 0:P] = at
            a_scr[i, P:2 * P] = at
        h_scr[g] = _dot_t(wemb_ref[...], featt_ref[g]) + bembt_ref[...]
        return 0

    jax.lax.fori_loop(0, NG, knn_body, 0)

    stats = None
    for l in range(N_LAYERS):
        w = _layer_w(wp_ref[l], bp_ref[l], mu_ref[l], is_ref[l])
        prev = stats

        def agg_body(g0, carry):
            cs, cq = carry
            for gg in range(UG):
                g = g0 * UG + gg
                h4 = h_scr[g]
                if prev is not None:
                    m, inv, gam, bet = prev
                    t = (y_scr[g] - m) * inv * gam + bet
                    h4 = h4 + jnp.maximum(t, 0.0)
                    h_scr[g] = h4
                yt4 = _agg_group_t(a_scr, g, h4, fcw_ref, w, l)
                y_scr[g] = yt4
                cs = cs + jnp.sum(yt4, axis=1, keepdims=True)
                cq = cq + jnp.sum(yt4 * yt4, axis=1, keepdims=True)
            return (cs, cq)

        zero = jnp.zeros((OUT, 1), F32)
        cs, cq = jax.lax.fori_loop(0, NG // UG, agg_body, (zero, zero + 0.0))

        n = np.float32(N)
        m = cs / n
        var = jnp.maximum(cq / n - m * m, 0.0)
        inv = jax.lax.rsqrt(var + np.float32(1e-5))
        stats = (m, inv, gamt_ref[l], bett_ref[l])

    m, inv, gam, bet = stats

    def read_body(g, _):
        t = (y_scr[g] - m) * inv * gam + bet
        hn4 = h_scr[g] + jnp.maximum(t, 0.0)
        cols = [jnp.sum(hn4[:, u * P:(u + 1) * P], axis=1, keepdims=True)
                * np.float32(1.0 / P) for u in range(JG)]
        hgt = jnp.transpose(jnp.concatenate(cols, axis=1))
        for u in range(JG):
            hg_ref[g * JG + u] = hgt[u:u + 1, :]
        return 0

    jax.lax.fori_loop(0, NG, read_body, 0)


def _mlp_krn(hg_ref, w0_ref, b0_ref, w1_ref, b1_ref, w2_ref, b2_ref, o_ref):
    x = jnp.maximum(jnp.dot(hg_ref[...], w0_ref[...],
                            preferred_element_type=F32) + b0_ref[...], 0.0)
    x = jnp.maximum(jnp.dot(x, w1_ref[...],
                            preferred_element_type=F32) + b1_ref[...], 0.0)
    o_ref[...] = jnp.dot(x, w2_ref[...],
                         preferred_element_type=F32) + b2_ref[...]


def _full_spec(shape):
    nd = len(shape)
    return pl.BlockSpec(shape, lambda *a: (0,) * nd)


def _sds(shape, dtype=F32):
    return jax.ShapeDtypeStruct(shape, dtype)


def kernel(points, features, lorentz_vectors, mask, params):
    del lorentz_vectors, mask
    layers = params['layers']

    wp_s = jnp.stack([lp['Wp'] for lp in layers])
    bp_s = jnp.stack([lp['bp'].reshape(1, DIM) for lp in layers])
    mu_s = jnp.stack([lp['mu'] for lp in layers])
    is_s = jnp.stack([lp['inv_sigma'] for lp in layers])
    fcw_s = jnp.concatenate(
        [lp['fcW'].reshape(HID, KERNEL, OUT).transpose(1, 0, 2)
         for lp in layers], axis=0)
    gam_s = jnp.stack([lp['gamma'].reshape(OUT, 1) for lp in layers])
    bet_s = jnp.stack([lp['beta'].reshape(OUT, 1) for lp in layers])

    featt = jnp.transpose(features.reshape(NG, JG, P, NUM_NODE_TYPE),
                          (0, 3, 1, 2)).reshape(NG, NUM_NODE_TYPE, PG)

    hg = pl.pallas_call(
        _monet_krn,
        in_specs=[_full_spec((B, 2, P)), _full_spec((NG, NUM_NODE_TYPE, PG)),
                  _full_spec((NUM_NODE_TYPE, HID)), _full_spec((HID, 1)),
                  _full_spec((N_LAYERS, 2, DIM)), _full_spec((N_LAYERS, 1, DIM)),
                  _full_spec((N_LAYERS, KERNEL, DIM)),
                  _full_spec((N_LAYERS, KERNEL, DIM)),
                  _full_spec((N_LAYERS * KERNEL, HID, OUT)),
                  _full_spec((N_LAYERS, OUT, 1)), _full_spec((N_LAYERS, OUT, 1))],
        out_specs=_full_spec((B, 1, OUT)),
        out_shape=_sds((B, 1, OUT)),
        scratch_shapes=[pltpu.VMEM((B, 2 * P, P), BF16),
                        pltpu.VMEM((NG, OUT, PG), F32),
                        pltpu.VMEM((NG, OUT, PG), F32)],
    )(jnp.transpose(points, (0, 2, 1)), featt,
      params['W_embed'], params['b_embed'].reshape(HID, 1),
      wp_s, bp_s, mu_s, is_s, fcw_s, gam_s, bet_s)

    mlp = params['mlp']
    out = pl.pallas_call(
        _mlp_krn,
        in_specs=[_full_spec((B, OUT)),
                  _full_spec((OUT, OUT // 2)), _full_spec((1, OUT // 2)),
                  _full_spec((OUT // 2, OUT // 4)), _full_spec((1, OUT // 4)),
                  _full_spec((OUT // 4, N_CLASSES)),
                  _full_spec((1, N_CLASSES))],
        out_specs=_full_spec((B, N_CLASSES)),
        out_shape=_sds((B, N_CLASSES)),
    )(hg.reshape(B, OUT), mlp['W0'], mlp['b0'].reshape(1, OUT // 2),
      mlp['W1'], mlp['b1'].reshape(1, OUT // 4),
      mlp['W2'], mlp['b2'].reshape(1, N_CLASSES))
    return out


# speedup vs baseline: 1.0609x; 1.0609x over previous
import jax
import jax.numpy as jnp
import numpy as np
from jax.experimental import pallas as pl
from jax.experimental.pallas import tpu as pltpu

B, P, K_NN = 256, 128, 16
NUM_NODE_TYPE, HID, OUT, N_CLASSES = 34, 70, 70, 5
KERNEL, DIM, N_LAYERS = 3, 2, 4
N = B * P
JG = 4
NG = B // JG
PG = JG * P
F32 = jnp.float32
BF16 = jnp.bfloat16
BIG = np.float32(3.0e38)
INV_K = np.float32(1.0 / K_NN)
UG = 4


def _knn_one_t(ptst):
    xr = ptst[0:1, :]
    yr = ptst[1:2, :]
    xrow = jnp.broadcast_to(xr, (P, P))
    yrow = jnp.broadcast_to(yr, (P, P))
    xcol = jnp.transpose(xrow)
    ycol = jnp.transpose(yrow)
    dx = xcol - xrow
    dy = ycol - yrow
    score = dx * dx + dy * dy
    row = jax.lax.broadcasted_iota(jnp.int32, (P, P), 0)

    def body(_, carry):
        score, acc = carry
        m = jnp.min(score, axis=0, keepdims=True)
        cidx = jnp.where(score == m, row, np.int32(2 ** 30))
        sel = jnp.min(cidx, axis=0, keepdims=True)
        pick = row == sel
        acc = acc + pick.astype(F32)
        score = jnp.where(pick, BIG, score)
        return score, acc

    _, acc = jax.lax.fori_loop(0, K_NN, body, (score, jnp.zeros_like(score)))
    return acc


def _layer_w(wp, bp, mu, iv):
    ones = (jax.lax.broadcasted_iota(jnp.int32, (1, DIM), 0) * 0 + 1).astype(F32)
    c = np.float32(1.0) / jnp.sqrt(ones * np.float32(K_NN + 1.0))
    ps = jnp.dot(c, wp, preferred_element_type=F32)
    pp = jnp.tanh(ps + bp)
    d = pp - mu
    gk = jnp.sum((np.float32(-0.5) * (d * d)) * (iv * iv),
                 axis=1, keepdims=True)
    return jnp.exp(gk)


def _dot_t(lhs, rhs):
    return jax.lax.dot_general(lhs, rhs, (((0,), (0,)), ((), ())),
                               preferred_element_type=F32)


def _agg_group_t(a_scr, g, h4, fcw_ref, w, layer):
    hkw = None
    for k in range(KERNEL):
        fck = fcw_ref[layer * KERNEL + k]
        hkt4 = _dot_t(fck, h4)
        t = hkt4 * w[k:k + 1, 0:1]
        hkw = t if hkw is None else hkw + t
    hi4 = hkw.astype(BF16)
    mid4 = (hkw - hi4.astype(F32)).astype(BF16)
    yjs = []
    for u in range(JG):
        sl = slice(u * P, (u + 1) * P)
        himid = jnp.concatenate([hi4[:, sl], mid4[:, sl]], axis=1)
        at2 = a_scr[g * JG + u]
        aggt = jnp.dot(himid, at2, preferred_element_type=F32)
        yjs.append(aggt * INV_K)
    return jnp.concatenate(yjs, axis=1)


def _monet_krn(ptst_ref, featt_ref, wemb_ref, bembt_ref, wp_ref, bp_ref,
               mu_ref, is_ref, fcw_ref, gamt_ref, bett_ref, hg_ref,
               a_scr, h_scr, y_scr):
    def knn_body(g, _):
        for u in range(JG):
            i = g * JG + u
            at = _knn_one_t(ptst_ref[i]).astype(BF16)
            a_scr[i, 0:P] = at
            a_scr[i, P:2 * P] = at
        h_scr[g] = _dot_t(wemb_ref[...], featt_ref[g]) + bembt_ref[...]
        return 0

    jax.lax.fori_loop(0, NG, knn_body, 0)

    stats = None
    for l in range(N_LAYERS):
        w = _layer_w(wp_ref[l], bp_ref[l], mu_ref[l], is_ref[l])
        prev = stats

        def agg_body(g0, carry):
            cs, cq = carry
            for gg in range(UG):
                g = g0 * UG + gg
                h4 = h_scr[g]
                if prev is not None:
                    m, inv, gam, bet = prev
                    t = (y_scr[g] - m) * inv * gam + bet
                    h4 = h4 + jnp.maximum(t, 0.0)
                    h_scr[g] = h4
                yt4 = _agg_group_t(a_scr, g, h4, fcw_ref, w, l)
                y_scr[g] = yt4
                cs = cs + jnp.sum(yt4, axis=1, keepdims=True)
                cq = cq + jnp.sum(yt4 * yt4, axis=1, keepdims=True)
            return (cs, cq)

        zero = jnp.zeros((OUT, 1), F32)
        cs, cq = jax.lax.fori_loop(0, NG // UG, agg_body, (zero, zero + 0.0))

        n = np.float32(N)
        m = cs / n
        var = jnp.maximum(cq / n - m * m, 0.0)
        inv = jax.lax.rsqrt(var + np.float32(1e-5))
        stats = (m, inv, gamt_ref[l], bett_ref[l])

    m, inv, gam, bet = stats

    def read_body(g, _):
        t = (y_scr[g] - m) * inv * gam + bet
        hn4 = h_scr[g] + jnp.maximum(t, 0.0)
        cols = [jnp.sum(hn4[:, u * P:(u + 1) * P], axis=1, keepdims=True)
                * np.float32(1.0 / P) for u in range(JG)]
        hgt = jnp.transpose(jnp.concatenate(cols, axis=1))
        for u in range(JG):
            hg_ref[g * JG + u] = hgt[u:u + 1, :]
        return 0

    jax.lax.fori_loop(0, NG, read_body, 0)


def _mlp_krn(hg_ref, w0_ref, b0_ref, w1_ref, b1_ref, w2_ref, b2_ref, o_ref):
    x = jnp.maximum(jnp.dot(hg_ref[...], w0_ref[...],
                            preferred_element_type=F32) + b0_ref[...], 0.0)
    x = jnp.maximum(jnp.dot(x, w1_ref[...],
                            preferred_element_type=F32) + b1_ref[...], 0.0)
    o_ref[...] = jnp.dot(x, w2_ref[...],
                         preferred_element_type=F32) + b2_ref[...]


def _full_spec(shape):
    nd = len(shape)
    return pl.BlockSpec(shape, lambda *a: (0,) * nd)


def _sds(shape, dtype=F32):
    return jax.ShapeDtypeStruct(shape, dtype)


def kernel(points, features, lorentz_vectors, mask, params):
    del lorentz_vectors, mask
    layers = params['layers']

    wp_s = jnp.stack([lp['Wp'] for lp in layers])
    bp_s = jnp.stack([lp['bp'].reshape(1, DIM) for lp in layers])
    mu_s = jnp.stack([lp['mu'] for lp in layers])
    is_s = jnp.stack([lp['inv_sigma'] for lp in layers])
    fcw_s = jnp.concatenate(
        [lp['fcW'].reshape(HID, KERNEL, OUT).transpose(1, 0, 2)
         for lp in layers], axis=0)
    gam_s = jnp.stack([lp['gamma'].reshape(OUT, 1) for lp in layers])
    bet_s = jnp.stack([lp['beta'].reshape(OUT, 1) for lp in layers])

    featt = jnp.transpose(features.reshape(NG, JG, P, NUM_NODE_TYPE),
                          (0, 3, 1, 2)).reshape(NG, NUM_NODE_TYPE, PG)

    hg = pl.pallas_call(
        _monet_krn,
        in_specs=[_full_spec((B, 2, P)), _full_spec((NG, NUM_NODE_TYPE, PG)),
                  _full_spec((NUM_NODE_TYPE, HID)), _full_spec((HID, 1)),
                  _full_spec((N_LAYERS, 2, DIM)), _full_spec((N_LAYERS, 1, DIM)),
                  _full_spec((N_LAYERS, KERNEL, DIM)),
                  _full_spec((N_LAYERS, KERNEL, DIM)),
                  _full_spec((N_LAYERS * KERNEL, HID, OUT)),
                  _full_spec((N_LAYERS, OUT, 1)), _full_spec((N_LAYERS, OUT, 1))],
        out_specs=_full_spec((B, 1, OUT)),
        out_shape=_sds((B, 1, OUT)),
        scratch_shapes=[pltpu.VMEM((B, 2 * P, P), BF16),
                        pltpu.VMEM((NG, OUT, PG), F32),
                        pltpu.VMEM((NG, OUT, PG), F32)],
    )(jnp.transpose(points, (0, 2, 1)), featt,
      params['W_embed'], params['b_embed'].reshape(HID, 1),
      wp_s, bp_s, mu_s, is_s, fcw_s, gam_s, bet_s)

    mlp = params['mlp']
    out = pl.pallas_call(
        _mlp_krn,
        in_specs=[_full_spec((B, OUT)),
                  _full_spec((OUT, OUT // 2)), _full_spec((1, OUT // 2)),
                  _full_spec((OUT // 2, OUT // 4)), _full_spec((1, OUT // 4)),
                  _full_spec((OUT // 4, N_CLASSES)),
                  _full_spec((1, N_CLASSES))],
        out_specs=_full_spec((B, N_CLASSES)),
        out_shape=_sds((B, N_CLASSES)),
    )(hg.reshape(B, OUT), mlp['W0'], mlp['b0'].reshape(1, OUT // 2),
      mlp['W1'], mlp['b1'].reshape(1, OUT // 4),
      mlp['W2'], mlp['b2'].reshape(1, N_CLASSES))
    return out


# speedup vs baseline: 1.1728x; 1.1055x over previous
import jax
import jax.numpy as jnp
import numpy as np
from jax.experimental import pallas as pl
from jax.experimental.pallas import tpu as pltpu

B, P, K_NN = 256, 128, 16
NUM_NODE_TYPE, HID, OUT, N_CLASSES = 34, 70, 70, 5
KERNEL, DIM, N_LAYERS = 3, 2, 4
N = B * P
JG = 8
NG = B // JG
PG = JG * P
F32 = jnp.float32
BF16 = jnp.bfloat16
BIG = np.float32(3.0e38)
INV_K = np.float32(1.0 / K_NN)
UG = 2


def _knn_one_t(ptst):
    xr = ptst[0:1, :]
    yr = ptst[1:2, :]
    xrow = jnp.broadcast_to(xr, (P, P))
    yrow = jnp.broadcast_to(yr, (P, P))
    xcol = jnp.transpose(xrow)
    ycol = jnp.transpose(yrow)
    dx = xcol - xrow
    dy = ycol - yrow
    score = dx * dx + dy * dy
    row = jax.lax.broadcasted_iota(jnp.int32, (P, P), 0)

    def body(_, carry):
        score, acc = carry
        m = jnp.min(score, axis=0, keepdims=True)
        cidx = jnp.where(score == m, row, np.int32(2 ** 30))
        sel = jnp.min(cidx, axis=0, keepdims=True)
        pick = row == sel
        acc = acc + pick.astype(F32)
        score = jnp.where(pick, BIG, score)
        return score, acc

    _, acc = jax.lax.fori_loop(0, K_NN, body, (score, jnp.zeros_like(score)))
    return acc


def _layer_w(wp, bp, mu, iv):
    ones = (jax.lax.broadcasted_iota(jnp.int32, (1, DIM), 0) * 0 + 1).astype(F32)
    c = np.float32(1.0) / jnp.sqrt(ones * np.float32(K_NN + 1.0))
    ps = jnp.dot(c, wp, preferred_element_type=F32)
    pp = jnp.tanh(ps + bp)
    d = pp - mu
    gk = jnp.sum((np.float32(-0.5) * (d * d)) * (iv * iv),
                 axis=1, keepdims=True)
    return jnp.exp(gk)


def _dot_t(lhs, rhs):
    return jax.lax.dot_general(lhs, rhs, (((0,), (0,)), ((), ())),
                               preferred_element_type=F32)


def _agg_group_t(a_scr, g, h4, fcw_ref, w, layer):
    hkw = None
    for k in range(KERNEL):
        fck = fcw_ref[layer * KERNEL + k]
        hkt4 = _dot_t(fck, h4)
        t = hkt4 * w[k:k + 1, 0:1]
        hkw = t if hkw is None else hkw + t
    hi4 = hkw.astype(BF16)
    mid4 = (hkw - hi4.astype(F32)).astype(BF16)
    yjs = []
    for u in range(JG):
        sl = slice(u * P, (u + 1) * P)
        himid = jnp.concatenate([hi4[:, sl], mid4[:, sl]], axis=1)
        at2 = a_scr[g * JG + u]
        aggt = jnp.dot(himid, at2, preferred_element_type=F32)
        yjs.append(aggt * INV_K)
    return jnp.concatenate(yjs, axis=1)


def _monet_krn(ptst_ref, featt_ref, wemb_ref, bembt_ref, wp_ref, bp_ref,
               mu_ref, is_ref, fcw_ref, gamt_ref, bett_ref, hg_ref,
               a_scr, h_scr, y_scr):
    def knn_body(g, _):
        for u in range(JG):
            i = g * JG + u
            at = _knn_one_t(ptst_ref[i]).astype(BF16)
            a_scr[i, 0:P] = at
            a_scr[i, P:2 * P] = at
        h_scr[g] = _dot_t(wemb_ref[...], featt_ref[g]) + bembt_ref[...]
        return 0

    jax.lax.fori_loop(0, NG, knn_body, 0)

    stats = None
    for l in range(N_LAYERS):
        w = _layer_w(wp_ref[l], bp_ref[l], mu_ref[l], is_ref[l])
        prev = stats

        def agg_body(g0, carry):
            cs, cq = carry
            for gg in range(UG):
                g = g0 * UG + gg
                h4 = h_scr[g]
                if prev is not None:
                    m, inv, gam, bet = prev
                    t = (y_scr[g] - m) * inv * gam + bet
                    h4 = h4 + jnp.maximum(t, 0.0)
                    h_scr[g] = h4
                yt4 = _agg_group_t(a_scr, g, h4, fcw_ref, w, l)
                y_scr[g] = yt4
                cs = cs + jnp.sum(yt4, axis=1, keepdims=True)
                cq = cq + jnp.sum(yt4 * yt4, axis=1, keepdims=True)
            return (cs, cq)

        zero = jnp.zeros((OUT, 1), F32)
        cs, cq = jax.lax.fori_loop(0, NG // UG, agg_body, (zero, zero + 0.0))

        n = np.float32(N)
        m = cs / n
        var = jnp.maximum(cq / n - m * m, 0.0)
        inv = jax.lax.rsqrt(var + np.float32(1e-5))
        stats = (m, inv, gamt_ref[l], bett_ref[l])

    m, inv, gam, bet = stats

    def read_body(g, _):
        t = (y_scr[g] - m) * inv * gam + bet
        hn4 = h_scr[g] + jnp.maximum(t, 0.0)
        cols = [jnp.sum(hn4[:, u * P:(u + 1) * P], axis=1, keepdims=True)
                * np.float32(1.0 / P) for u in range(JG)]
        hgt = jnp.transpose(jnp.concatenate(cols, axis=1))
        for u in range(JG):
            hg_ref[g * JG + u] = hgt[u:u + 1, :]
        return 0

    jax.lax.fori_loop(0, NG, read_body, 0)


def _mlp_krn(hg_ref, w0_ref, b0_ref, w1_ref, b1_ref, w2_ref, b2_ref, o_ref):
    x = jnp.maximum(jnp.dot(hg_ref[...], w0_ref[...],
                            preferred_element_type=F32) + b0_ref[...], 0.0)
    x = jnp.maximum(jnp.dot(x, w1_ref[...],
                            preferred_element_type=F32) + b1_ref[...], 0.0)
    o_ref[...] = jnp.dot(x, w2_ref[...],
                         preferred_element_type=F32) + b2_ref[...]


def _full_spec(shape):
    nd = len(shape)
    return pl.BlockSpec(shape, lambda *a: (0,) * nd)


def _sds(shape, dtype=F32):
    return jax.ShapeDtypeStruct(shape, dtype)


def kernel(points, features, lorentz_vectors, mask, params):
    del lorentz_vectors, mask
    layers = params['layers']

    wp_s = jnp.stack([lp['Wp'] for lp in layers])
    bp_s = jnp.stack([lp['bp'].reshape(1, DIM) for lp in layers])
    mu_s = jnp.stack([lp['mu'] for lp in layers])
    is_s = jnp.stack([lp['inv_sigma'] for lp in layers])
    fcw_s = jnp.concatenate(
        [lp['fcW'].reshape(HID, KERNEL, OUT).transpose(1, 0, 2)
         for lp in layers], axis=0)
    gam_s = jnp.stack([lp['gamma'].reshape(OUT, 1) for lp in layers])
    bet_s = jnp.stack([lp['beta'].reshape(OUT, 1) for lp in layers])

    featt = jnp.transpose(features.reshape(NG, JG, P, NUM_NODE_TYPE),
                          (0, 3, 1, 2)).reshape(NG, NUM_NODE_TYPE, PG)

    hg = pl.pallas_call(
        _monet_krn,
        in_specs=[_full_spec((B, 2, P)), _full_spec((NG, NUM_NODE_TYPE, PG)),
                  _full_spec((NUM_NODE_TYPE, HID)), _full_spec((HID, 1)),
                  _full_spec((N_LAYERS, 2, DIM)), _full_spec((N_LAYERS, 1, DIM)),
                  _full_spec((N_LAYERS, KERNEL, DIM)),
                  _full_spec((N_LAYERS, KERNEL, DIM)),
                  _full_spec((N_LAYERS * KERNEL, HID, OUT)),
                  _full_spec((N_LAYERS, OUT, 1)), _full_spec((N_LAYERS, OUT, 1))],
        out_specs=_full_spec((B, 1, OUT)),
        out_shape=_sds((B, 1, OUT)),
        scratch_shapes=[pltpu.VMEM((B, 2 * P, P), BF16),
                        pltpu.VMEM((NG, OUT, PG), F32),
                        pltpu.VMEM((NG, OUT, PG), F32)],
    )(jnp.transpose(points, (0, 2, 1)), featt,
      params['W_embed'], params['b_embed'].reshape(HID, 1),
      wp_s, bp_s, mu_s, is_s, fcw_s, gam_s, bet_s)

    mlp = params['mlp']
    out = pl.pallas_call(
        _mlp_krn,
        in_specs=[_full_spec((B, OUT)),
                  _full_spec((OUT, OUT // 2)), _full_spec((1, OUT // 2)),
                  _full_spec((OUT // 2, OUT // 4)), _full_spec((1, OUT // 4)),
                  _full_spec((OUT // 4, N_CLASSES)),
                  _full_spec((1, N_CLASSES))],
        out_specs=_full_spec((B, N_CLASSES)),
        out_shape=_sds((B, N_CLASSES)),
    )(hg.reshape(B, OUT), mlp['W0'], mlp['b0'].reshape(1, OUT // 2),
      mlp['W1'], mlp['b1'].reshape(1, OUT // 4),
      mlp['W2'], mlp['b2'].reshape(1, N_CLASSES))
    return out


# speedup vs baseline: 1.2850x; 1.0956x over previous
import jax
import jax.numpy as jnp
import numpy as np
from jax.experimental import pallas as pl
from jax.experimental.pallas import tpu as pltpu

B, P, K_NN = 256, 128, 16
NUM_NODE_TYPE, HID, OUT, N_CLASSES = 34, 70, 70, 5
KERNEL, DIM, N_LAYERS = 3, 2, 4
N = B * P
JG = 16
NG = B // JG
PG = JG * P
F32 = jnp.float32
BF16 = jnp.bfloat16
BIG = np.float32(3.0e38)
INV_K = np.float32(1.0 / K_NN)
UG = 2


def _knn_one_t(ptst):
    xr = ptst[0:1, :]
    yr = ptst[1:2, :]
    xrow = jnp.broadcast_to(xr, (P, P))
    yrow = jnp.broadcast_to(yr, (P, P))
    xcol = jnp.transpose(xrow)
    ycol = jnp.transpose(yrow)
    dx = xcol - xrow
    dy = ycol - yrow
    score = dx * dx + dy * dy
    row = jax.lax.broadcasted_iota(jnp.int32, (P, P), 0)

    def body(_, carry):
        score, acc = carry
        m = jnp.min(score, axis=0, keepdims=True)
        cidx = jnp.where(score == m, row, np.int32(2 ** 30))
        sel = jnp.min(cidx, axis=0, keepdims=True)
        pick = row == sel
        acc = acc + pick.astype(F32)
        score = jnp.where(pick, BIG, score)
        return score, acc

    _, acc = jax.lax.fori_loop(0, K_NN, body, (score, jnp.zeros_like(score)))
    return acc


def _layer_w(wp, bp, mu, iv):
    ones = (jax.lax.broadcasted_iota(jnp.int32, (1, DIM), 0) * 0 + 1).astype(F32)
    c = np.float32(1.0) / jnp.sqrt(ones * np.float32(K_NN + 1.0))
    ps = jnp.dot(c, wp, preferred_element_type=F32)
    pp = jnp.tanh(ps + bp)
    d = pp - mu
    gk = jnp.sum((np.float32(-0.5) * (d * d)) * (iv * iv),
                 axis=1, keepdims=True)
    return jnp.exp(gk)


def _dot_t(lhs, rhs):
    return jax.lax.dot_general(lhs, rhs, (((0,), (0,)), ((), ())),
                               preferred_element_type=F32)


def _agg_group_t(a_scr, g, h4, fcw_ref, w, layer):
    hkw = None
    for k in range(KERNEL):
        fck = fcw_ref[layer * KERNEL + k]
        hkt4 = _dot_t(fck, h4)
        t = hkt4 * w[k:k + 1, 0:1]
        hkw = t if hkw is None else hkw + t
    hi4 = hkw.astype(BF16)
    mid4 = (hkw - hi4.astype(F32)).astype(BF16)
    yjs = []
    for u in range(JG):
        sl = slice(u * P, (u + 1) * P)
        himid = jnp.concatenate([hi4[:, sl], mid4[:, sl]], axis=1)
        at2 = a_scr[g * JG + u]
        aggt = jnp.dot(himid, at2, preferred_element_type=F32)
        yjs.append(aggt * INV_K)
    return jnp.concatenate(yjs, axis=1)


def _monet_krn(ptst_ref, featt_ref, wemb_ref, bembt_ref, wp_ref, bp_ref,
               mu_ref, is_ref, fcw_ref, gamt_ref, bett_ref, hg_ref,
               a_scr, h_scr, y_scr):
    def knn_body(g, _):
        for u in range(JG):
            i = g * JG + u
            at = _knn_one_t(ptst_ref[i]).astype(BF16)
            a_scr[i, 0:P] = at
            a_scr[i, P:2 * P] = at
        h_scr[g] = _dot_t(wemb_ref[...], featt_ref[g]) + bembt_ref[...]
        return 0

    jax.lax.fori_loop(0, NG, knn_body, 0)

    stats = None
    for l in range(N_LAYERS):
        w = _layer_w(wp_ref[l], bp_ref[l], mu_ref[l], is_ref[l])
        prev = stats

        def agg_body(g0, carry):
            cs, cq = carry
            for gg in range(UG):
                g = g0 * UG + gg
                h4 = h_scr[g]
                if prev is not None:
                    m, inv, gam, bet = prev
                    t = (y_scr[g] - m) * inv * gam + bet
                    h4 = h4 + jnp.maximum(t, 0.0)
                    h_scr[g] = h4
                yt4 = _agg_group_t(a_scr, g, h4, fcw_ref, w, l)
                y_scr[g] = yt4
                cs = cs + jnp.sum(yt4, axis=1, keepdims=True)
                cq = cq + jnp.sum(yt4 * yt4, axis=1, keepdims=True)
            return (cs, cq)

        zero = jnp.zeros((OUT, 1), F32)
        cs, cq = jax.lax.fori_loop(0, NG // UG, agg_body, (zero, zero + 0.0))

        n = np.float32(N)
        m = cs / n
        var = jnp.maximum(cq / n - m * m, 0.0)
        inv = jax.lax.rsqrt(var + np.float32(1e-5))
        stats = (m, inv, gamt_ref[l], bett_ref[l])

    m, inv, gam, bet = stats

    def read_body(g, _):
        t = (y_scr[g] - m) * inv * gam + bet
        hn4 = h_scr[g] + jnp.maximum(t, 0.0)
        cols = [jnp.sum(hn4[:, u * P:(u + 1) * P], axis=1, keepdims=True)
                * np.float32(1.0 / P) for u in range(JG)]
        hgt = jnp.transpose(jnp.concatenate(cols, axis=1))
        for u in range(JG):
            hg_ref[g * JG + u] = hgt[u:u + 1, :]
        return 0

    jax.lax.fori_loop(0, NG, read_body, 0)


def _mlp_krn(hg_ref, w0_ref, b0_ref, w1_ref, b1_ref, w2_ref, b2_ref, o_ref):
    x = jnp.maximum(jnp.dot(hg_ref[...], w0_ref[...],
                            preferred_element_type=F32) + b0_ref[...], 0.0)
    x = jnp.maximum(jnp.dot(x, w1_ref[...],
                            preferred_element_type=F32) + b1_ref[...], 0.0)
    o_ref[...] = jnp.dot(x, w2_ref[...],
                         preferred_element_type=F32) + b2_ref[...]


def _full_spec(shape):
    nd = len(shape)
    return pl.BlockSpec(shape, lambda *a: (0,) * nd)


def _sds(shape, dtype=F32):
    return jax.ShapeDtypeStruct(shape, dtype)


def kernel(points, features, lorentz_vectors, mask, params):
    del lorentz_vectors, mask
    layers = params['layers']

    wp_s = jnp.stack([lp['Wp'] for lp in layers])
    bp_s = jnp.stack([lp['bp'].reshape(1, DIM) for lp in layers])
    mu_s = jnp.stack([lp['mu'] for lp in layers])
    is_s = jnp.stack([lp['inv_sigma'] for lp in layers])
    fcw_s = jnp.concatenate(
        [lp['fcW'].reshape(HID, KERNEL, OUT).transpose(1, 0, 2)
         for lp in layers], axis=0)
    gam_s = jnp.stack([lp['gamma'].reshape(OUT, 1) for lp in layers])
    bet_s = jnp.stack([lp['beta'].reshape(OUT, 1) for lp in layers])

    featt = jnp.transpose(features.reshape(NG, JG, P, NUM_NODE_TYPE),
                          (0, 3, 1, 2)).reshape(NG, NUM_NODE_TYPE, PG)

    hg = pl.pallas_call(
        _monet_krn,
        in_specs=[_full_spec((B, 2, P)), _full_spec((NG, NUM_NODE_TYPE, PG)),
                  _full_spec((NUM_NODE_TYPE, HID)), _full_spec((HID, 1)),
                  _full_spec((N_LAYERS, 2, DIM)), _full_spec((N_LAYERS, 1, DIM)),
                  _full_spec((N_LAYERS, KERNEL, DIM)),
                  _full_spec((N_LAYERS, KERNEL, DIM)),
                  _full_spec((N_LAYERS * KERNEL, HID, OUT)),
                  _full_spec((N_LAYERS, OUT, 1)), _full_spec((N_LAYERS, OUT, 1))],
        out_specs=_full_spec((B, 1, OUT)),
        out_shape=_sds((B, 1, OUT)),
        scratch_shapes=[pltpu.VMEM((B, 2 * P, P), BF16),
                        pltpu.VMEM((NG, OUT, PG), F32),
                        pltpu.VMEM((NG, OUT, PG), F32)],
    )(jnp.transpose(points, (0, 2, 1)), featt,
      params['W_embed'], params['b_embed'].reshape(HID, 1),
      wp_s, bp_s, mu_s, is_s, fcw_s, gam_s, bet_s)

    mlp = params['mlp']
    out = pl.pallas_call(
        _mlp_krn,
        in_specs=[_full_spec((B, OUT)),
                  _full_spec((OUT, OUT // 2)), _full_spec((1, OUT // 2)),
                  _full_spec((OUT // 2, OUT // 4)), _full_spec((1, OUT // 4)),
                  _full_spec((OUT // 4, N_CLASSES)),
                  _full_spec((1, N_CLASSES))],
        out_specs=_full_spec((B, N_CLASSES)),
        out_shape=_sds((B, N_CLASSES)),
    )(hg.reshape(B, OUT), mlp['W0'], mlp['b0'].reshape(1, OUT // 2),
      mlp['W1'], mlp['b1'].reshape(1, OUT // 4),
      mlp['W2'], mlp['b2'].reshape(1, N_CLASSES))
    return out


# speedup vs baseline: 1.3479x; 1.0490x over previous
import jax
import jax.numpy as jnp
import numpy as np
from jax.experimental import pallas as pl
from jax.experimental.pallas import tpu as pltpu

B, P, K_NN = 256, 128, 16
NUM_NODE_TYPE, HID, OUT, N_CLASSES = 34, 70, 70, 5
KERNEL, DIM, N_LAYERS = 3, 2, 4
N = B * P
JG = 32
NG = B // JG
PG = JG * P
F32 = jnp.float32
BF16 = jnp.bfloat16
BIG = np.float32(3.0e38)
INV_K = np.float32(1.0 / K_NN)
UG = 2


def _knn_one_t(ptst):
    xr = ptst[0:1, :]
    yr = ptst[1:2, :]
    xrow = jnp.broadcast_to(xr, (P, P))
    yrow = jnp.broadcast_to(yr, (P, P))
    xcol = jnp.transpose(xrow)
    ycol = jnp.transpose(yrow)
    dx = xcol - xrow
    dy = ycol - yrow
    score = dx * dx + dy * dy
    row = jax.lax.broadcasted_iota(jnp.int32, (P, P), 0)

    def body(_, carry):
        score, acc = carry
        m = jnp.min(score, axis=0, keepdims=True)
        cidx = jnp.where(score == m, row, np.int32(2 ** 30))
        sel = jnp.min(cidx, axis=0, keepdims=True)
        pick = row == sel
        acc = acc + pick.astype(F32)
        score = jnp.where(pick, BIG, score)
        return score, acc

    _, acc = jax.lax.fori_loop(0, K_NN, body, (score, jnp.zeros_like(score)))
    return acc


def _layer_w(wp, bp, mu, iv):
    ones = (jax.lax.broadcasted_iota(jnp.int32, (1, DIM), 0) * 0 + 1).astype(F32)
    c = np.float32(1.0) / jnp.sqrt(ones * np.float32(K_NN + 1.0))
    ps = jnp.dot(c, wp, preferred_element_type=F32)
    pp = jnp.tanh(ps + bp)
    d = pp - mu
    gk = jnp.sum((np.float32(-0.5) * (d * d)) * (iv * iv),
                 axis=1, keepdims=True)
    return jnp.exp(gk)


def _dot_t(lhs, rhs):
    return jax.lax.dot_general(lhs, rhs, (((0,), (0,)), ((), ())),
                               preferred_element_type=F32)


def _agg_group_t(a_scr, g, h4, fcw_ref, w, layer):
    hkw = None
    for k in range(KERNEL):
        fck = fcw_ref[layer * KERNEL + k]
        hkt4 = _dot_t(fck, h4)
        t = hkt4 * w[k:k + 1, 0:1]
        hkw = t if hkw is None else hkw + t
    hi4 = hkw.astype(BF16)
    mid4 = (hkw - hi4.astype(F32)).astype(BF16)
    yjs = []
    for u in range(JG):
        sl = slice(u * P, (u + 1) * P)
        himid = jnp.concatenate([hi4[:, sl], mid4[:, sl]], axis=1)
        at2 = a_scr[g * JG + u]
        aggt = jnp.dot(himid, at2, preferred_element_type=F32)
        yjs.append(aggt * INV_K)
    return jnp.concatenate(yjs, axis=1)


def _monet_krn(ptst_ref, featt_ref, wemb_ref, bembt_ref, wp_ref, bp_ref,
               mu_ref, is_ref, fcw_ref, gamt_ref, bett_ref, hg_ref,
               a_scr, h_scr, y_scr):
    def knn_body(g, _):
        for u in range(JG):
            i = g * JG + u
            at = _knn_one_t(ptst_ref[i]).astype(BF16)
            a_scr[i, 0:P] = at
            a_scr[i, P:2 * P] = at
        h_scr[g] = _dot_t(wemb_ref[...], featt_ref[g]) + bembt_ref[...]
        return 0

    jax.lax.fori_loop(0, NG, knn_body, 0)

    stats = None
    for l in range(N_LAYERS):
        w = _layer_w(wp_ref[l], bp_ref[l], mu_ref[l], is_ref[l])
        prev = stats

        def agg_body(g0, carry):
            cs, cq = carry
            for gg in range(UG):
                g = g0 * UG + gg
                h4 = h_scr[g]
                if prev is not None:
                    m, inv, gam, bet = prev
                    t = (y_scr[g] - m) * inv * gam + bet
                    h4 = h4 + jnp.maximum(t, 0.0)
                    h_scr[g] = h4
                yt4 = _agg_group_t(a_scr, g, h4, fcw_ref, w, l)
                y_scr[g] = yt4
                cs = cs + jnp.sum(yt4, axis=1, keepdims=True)
                cq = cq + jnp.sum(yt4 * yt4, axis=1, keepdims=True)
            return (cs, cq)

        zero = jnp.zeros((OUT, 1), F32)
        cs, cq = jax.lax.fori_loop(0, NG // UG, agg_body, (zero, zero + 0.0))

        n = np.float32(N)
        m = cs / n
        var = jnp.maximum(cq / n - m * m, 0.0)
        inv = jax.lax.rsqrt(var + np.float32(1e-5))
        stats = (m, inv, gamt_ref[l], bett_ref[l])

    m, inv, gam, bet = stats

    def read_body(g, _):
        t = (y_scr[g] - m) * inv * gam + bet
        hn4 = h_scr[g] + jnp.maximum(t, 0.0)
        cols = [jnp.sum(hn4[:, u * P:(u + 1) * P], axis=1, keepdims=True)
                * np.float32(1.0 / P) for u in range(JG)]
        hgt = jnp.transpose(jnp.concatenate(cols, axis=1))
        for u in range(JG):
            hg_ref[g * JG + u] = hgt[u:u + 1, :]
        return 0

    jax.lax.fori_loop(0, NG, read_body, 0)


def _mlp_krn(hg_ref, w0_ref, b0_ref, w1_ref, b1_ref, w2_ref, b2_ref, o_ref):
    x = jnp.maximum(jnp.dot(hg_ref[...], w0_ref[...],
                            preferred_element_type=F32) + b0_ref[...], 0.0)
    x = jnp.maximum(jnp.dot(x, w1_ref[...],
                            preferred_element_type=F32) + b1_ref[...], 0.0)
    o_ref[...] = jnp.dot(x, w2_ref[...],
                         preferred_element_type=F32) + b2_ref[...]


def _full_spec(shape):
    nd = len(shape)
    return pl.BlockSpec(shape, lambda *a: (0,) * nd)


def _sds(shape, dtype=F32):
    return jax.ShapeDtypeStruct(shape, dtype)


def kernel(points, features, lorentz_vectors, mask, params):
    del lorentz_vectors, mask
    layers = params['layers']

    wp_s = jnp.stack([lp['Wp'] for lp in layers])
    bp_s = jnp.stack([lp['bp'].reshape(1, DIM) for lp in layers])
    mu_s = jnp.stack([lp['mu'] for lp in layers])
    is_s = jnp.stack([lp['inv_sigma'] for lp in layers])
    fcw_s = jnp.concatenate(
        [lp['fcW'].reshape(HID, KERNEL, OUT).transpose(1, 0, 2)
         for lp in layers], axis=0)
    gam_s = jnp.stack([lp['gamma'].reshape(OUT, 1) for lp in layers])
    bet_s = jnp.stack([lp['beta'].reshape(OUT, 1) for lp in layers])

    featt = jnp.transpose(features.reshape(NG, JG, P, NUM_NODE_TYPE),
                          (0, 3, 1, 2)).reshape(NG, NUM_NODE_TYPE, PG)

    hg = pl.pallas_call(
        _monet_krn,
        in_specs=[_full_spec((B, 2, P)), _full_spec((NG, NUM_NODE_TYPE, PG)),
                  _full_spec((NUM_NODE_TYPE, HID)), _full_spec((HID, 1)),
                  _full_spec((N_LAYERS, 2, DIM)), _full_spec((N_LAYERS, 1, DIM)),
                  _full_spec((N_LAYERS, KERNEL, DIM)),
                  _full_spec((N_LAYERS, KERNEL, DIM)),
                  _full_spec((N_LAYERS * KERNEL, HID, OUT)),
                  _full_spec((N_LAYERS, OUT, 1)), _full_spec((N_LAYERS, OUT, 1))],
        out_specs=_full_spec((B, 1, OUT)),
        out_shape=_sds((B, 1, OUT)),
        scratch_shapes=[pltpu.VMEM((B, 2 * P, P), BF16),
                        pltpu.VMEM((NG, OUT, PG), F32),
                        pltpu.VMEM((NG, OUT, PG), F32)],
    )(jnp.transpose(points, (0, 2, 1)), featt,
      params['W_embed'], params['b_embed'].reshape(HID, 1),
      wp_s, bp_s, mu_s, is_s, fcw_s, gam_s, bet_s)

    mlp = params['mlp']
    out = pl.pallas_call(
        _mlp_krn,
        in_specs=[_full_spec((B, OUT)),
                  _full_spec((OUT, OUT // 2)), _full_spec((1, OUT // 2)),
                  _full_spec((OUT // 2, OUT // 4)), _full_spec((1, OUT // 4)),
                  _full_spec((OUT // 4, N_CLASSES)),
                  _full_spec((1, N_CLASSES))],
        out_specs=_full_spec((B, N_CLASSES)),
        out_shape=_sds((B, N_CLASSES)),
    )(hg.reshape(B, OUT), mlp['W0'], mlp['b0'].reshape(1, OUT // 2),
      mlp['W1'], mlp['b1'].reshape(1, OUT // 4),
      mlp['W2'], mlp['b2'].reshape(1, N_CLASSES))
    return out


# speedup vs baseline: 1.4707x; 1.0911x over previous
import jax
import jax.numpy as jnp
import numpy as np
from jax.experimental import pallas as pl
from jax.experimental.pallas import tpu as pltpu

B, P, K_NN = 256, 128, 16
NUM_NODE_TYPE, HID, OUT, N_CLASSES = 34, 70, 70, 5
KERNEL, DIM, N_LAYERS = 3, 2, 4
N = B * P
JG = 32
NG = B // JG
PG = JG * P
F32 = jnp.float32
BF16 = jnp.bfloat16
BIG = np.float32(3.0e38)
INV_K = np.float32(1.0 / K_NN)
UG = 2


def _knn_score_t(ptst):
    xr = ptst[0:1, :]
    yr = ptst[1:2, :]
    xrow = jnp.broadcast_to(xr, (P, P))
    yrow = jnp.broadcast_to(yr, (P, P))
    xcol = jnp.transpose(xrow)
    ycol = jnp.transpose(yrow)
    dx = xcol - xrow
    dy = ycol - yrow
    return dx * dx + dy * dy


def _knn_pair_t(pa, pb):
    sa = _knn_score_t(pa)
    sb = _knn_score_t(pb)
    row = jax.lax.broadcasted_iota(jnp.int32, (P, P), 0)

    def pick_one(score, acc):
        m = jnp.min(score, axis=0, keepdims=True)
        cidx = jnp.where(score == m, row, np.int32(2 ** 30))
        sel = jnp.min(cidx, axis=0, keepdims=True)
        pick = row == sel
        acc = acc + pick.astype(F32)
        score = jnp.where(pick, BIG, score)
        return score, acc

    def body(_, carry):
        sa, aa, sb, ab = carry
        sa, aa = pick_one(sa, aa)
        sb, ab = pick_one(sb, ab)
        return sa, aa, sb, ab

    _, aa, _, ab = jax.lax.fori_loop(
        0, K_NN, body,
        (sa, jnp.zeros_like(sa), sb, jnp.zeros_like(sb)))
    return aa, ab


def _layer_w(wp, bp, mu, iv):
    ones = (jax.lax.broadcasted_iota(jnp.int32, (1, DIM), 0) * 0 + 1).astype(F32)
    c = np.float32(1.0) / jnp.sqrt(ones * np.float32(K_NN + 1.0))
    ps = jnp.dot(c, wp, preferred_element_type=F32)
    pp = jnp.tanh(ps + bp)
    d = pp - mu
    gk = jnp.sum((np.float32(-0.5) * (d * d)) * (iv * iv),
                 axis=1, keepdims=True)
    return jnp.exp(gk)


def _dot_t(lhs, rhs):
    return jax.lax.dot_general(lhs, rhs, (((0,), (0,)), ((), ())),
                               preferred_element_type=F32)


def _agg_group_t(a_scr, g, h4, fcw_ref, w, layer):
    hkw = None
    for k in range(KERNEL):
        fck = fcw_ref[layer * KERNEL + k]
        hkt4 = _dot_t(fck, h4)
        t = hkt4 * w[k:k + 1, 0:1]
        hkw = t if hkw is None else hkw + t
    hi4 = hkw.astype(BF16)
    mid4 = (hkw - hi4.astype(F32)).astype(BF16)
    yjs = []
    for u in range(JG):
        sl = slice(u * P, (u + 1) * P)
        himid = jnp.concatenate([hi4[:, sl], mid4[:, sl]], axis=1)
        at2 = a_scr[g * JG + u]
        aggt = jnp.dot(himid, at2, preferred_element_type=F32)
        yjs.append(aggt * INV_K)
    return jnp.concatenate(yjs, axis=1)


def _monet_krn(ptst_ref, featt_ref, wemb_ref, bembt_ref, wp_ref, bp_ref,
               mu_ref, is_ref, fcw_ref, gamt_ref, bett_ref, hg_ref,
               a_scr, h_scr, y_scr):
    def knn_body(g, _):
        for u in range(0, JG, 2):
            i = g * JG + u
            ata, atb = _knn_pair_t(ptst_ref[i], ptst_ref[i + 1])
            a_scr[i, 0:P] = ata.astype(BF16)
            a_scr[i, P:2 * P] = ata.astype(BF16)
            a_scr[i + 1, 0:P] = atb.astype(BF16)
            a_scr[i + 1, P:2 * P] = atb.astype(BF16)
        h_scr[g] = _dot_t(wemb_ref[...], featt_ref[g]) + bembt_ref[...]
        return 0

    jax.lax.fori_loop(0, NG, knn_body, 0)

    stats = None
    for l in range(N_LAYERS):
        w = _layer_w(wp_ref[l], bp_ref[l], mu_ref[l], is_ref[l])
        prev = stats

        def agg_body(g0, carry):
            cs, cq = carry
            for gg in range(UG):
                g = g0 * UG + gg
                h4 = h_scr[g]
                if prev is not None:
                    m, inv, gam, bet = prev
                    t = (y_scr[g] - m) * inv * gam + bet
                    h4 = h4 + jnp.maximum(t, 0.0)
                    h_scr[g] = h4
                yt4 = _agg_group_t(a_scr, g, h4, fcw_ref, w, l)
                y_scr[g] = yt4
                cs = cs + jnp.sum(yt4, axis=1, keepdims=True)
                cq = cq + jnp.sum(yt4 * yt4, axis=1, keepdims=True)
            return (cs, cq)

        zero = jnp.zeros((OUT, 1), F32)
        cs, cq = jax.lax.fori_loop(0, NG // UG, agg_body, (zero, zero + 0.0))

        n = np.float32(N)
        m = cs / n
        var = jnp.maximum(cq / n - m * m, 0.0)
        inv = jax.lax.rsqrt(var + np.float32(1e-5))
        stats = (m, inv, gamt_ref[l], bett_ref[l])

    m, inv, gam, bet = stats

    def read_body(g, _):
        t = (y_scr[g] - m) * inv * gam + bet
        hn4 = h_scr[g] + jnp.maximum(t, 0.0)
        cols = [jnp.sum(hn4[:, u * P:(u + 1) * P], axis=1, keepdims=True)
                * np.float32(1.0 / P) for u in range(JG)]
        hgt = jnp.transpose(jnp.concatenate(cols, axis=1))
        for u in range(JG):
            hg_ref[g * JG + u] = hgt[u:u + 1, :]
        return 0

    jax.lax.fori_loop(0, NG, read_body, 0)


def _mlp_krn(hg_ref, w0_ref, b0_ref, w1_ref, b1_ref, w2_ref, b2_ref, o_ref):
    x = jnp.maximum(jnp.dot(hg_ref[...], w0_ref[...],
                            preferred_element_type=F32) + b0_ref[...], 0.0)
    x = jnp.maximum(jnp.dot(x, w1_ref[...],
                            preferred_element_type=F32) + b1_ref[...], 0.0)
    o_ref[...] = jnp.dot(x, w2_ref[...],
                         preferred_element_type=F32) + b2_ref[...]


def _full_spec(shape):
    nd = len(shape)
    return pl.BlockSpec(shape, lambda *a: (0,) * nd)


def _sds(shape, dtype=F32):
    return jax.ShapeDtypeStruct(shape, dtype)


def kernel(points, features, lorentz_vectors, mask, params):
    del lorentz_vectors, mask
    layers = params['layers']

    wp_s = jnp.stack([lp['Wp'] for lp in layers])
    bp_s = jnp.stack([lp['bp'].reshape(1, DIM) for lp in layers])
    mu_s = jnp.stack([lp['mu'] for lp in layers])
    is_s = jnp.stack([lp['inv_sigma'] for lp in layers])
    fcw_s = jnp.concatenate(
        [lp['fcW'].reshape(HID, KERNEL, OUT).transpose(1, 0, 2)
         for lp in layers], axis=0)
    gam_s = jnp.stack([lp['gamma'].reshape(OUT, 1) for lp in layers])
    bet_s = jnp.stack([lp['beta'].reshape(OUT, 1) for lp in layers])

    featt = jnp.transpose(features.reshape(NG, JG, P, NUM_NODE_TYPE),
                          (0, 3, 1, 2)).reshape(NG, NUM_NODE_TYPE, PG)

    hg = pl.pallas_call(
        _monet_krn,
        in_specs=[_full_spec((B, 2, P)), _full_spec((NG, NUM_NODE_TYPE, PG)),
                  _full_spec((NUM_NODE_TYPE, HID)), _full_spec((HID, 1)),
                  _full_spec((N_LAYERS, 2, DIM)), _full_spec((N_LAYERS, 1, DIM)),
                  _full_spec((N_LAYERS, KERNEL, DIM)),
                  _full_spec((N_LAYERS, KERNEL, DIM)),
                  _full_spec((N_LAYERS * KERNEL, HID, OUT)),
                  _full_spec((N_LAYERS, OUT, 1)), _full_spec((N_LAYERS, OUT, 1))],
        out_specs=_full_spec((B, 1, OUT)),
        out_shape=_sds((B, 1, OUT)),
        scratch_shapes=[pltpu.VMEM((B, 2 * P, P), BF16),
                        pltpu.VMEM((NG, OUT, PG), F32),
                        pltpu.VMEM((NG, OUT, PG), F32)],
    )(jnp.transpose(points, (0, 2, 1)), featt,
      params['W_embed'], params['b_embed'].reshape(HID, 1),
      wp_s, bp_s, mu_s, is_s, fcw_s, gam_s, bet_s)

    mlp = params['mlp']
    out = pl.pallas_call(
        _mlp_krn,
        in_specs=[_full_spec((B, OUT)),
                  _full_spec((OUT, OUT // 2)), _full_spec((1, OUT // 2)),
                  _full_spec((OUT // 2, OUT // 4)), _full_spec((1, OUT // 4)),
                  _full_spec((OUT // 4, N_CLASSES)),
                  _full_spec((1, N_CLASSES))],
        out_specs=_full_spec((B, N_CLASSES)),
        out_shape=_sds((B, N_CLASSES)),
    )(hg.reshape(B, OUT), mlp['W0'], mlp['b0'].reshape(1, OUT // 2),
      mlp['W1'], mlp['b1'].reshape(1, OUT // 4),
      mlp['W2'], mlp['b2'].reshape(1, N_CLASSES))
    return out
